# baseline (speedup 1.0000x reference)
import functools
from math import sqrt

import jax
import jax.numpy as jnp
from jax.experimental import pallas as pl
from jax.experimental.pallas import tpu as pltpu

LOCAL_HALF = 8
SP1 = 65


def _strided_body(q_ref, k_ref, v_ref, num_ref, den_ref, *, G, A, scale):
    qb = q_ref[0]
    kb = k_ref[0]
    vb = v_ref[0]
    s = jax.lax.dot_general(qb, kb, (((1,), (1,)), ((), ())),
                            preferred_element_type=jnp.float32)
    n = G * A
    ri = jax.lax.broadcasted_iota(jnp.int32, (n, n), 0)
    ci = jax.lax.broadcasted_iota(jnp.int32, (n, n), 1)
    same_r = (ri // A) == (ci // A)
    lower = (ci % A) < (ri % A)
    w = jnp.where(same_r & lower, jnp.exp(scale * s) - 1.0, 0.0)
    num_ref[0] = jax.lax.dot(w, vb, preferred_element_type=jnp.float32)
    den_ref[0] = jnp.sum(w, axis=1, keepdims=True)


def _local_body(q_ref, k_ref, v_ref, nb_ref, db_ref, o_ref, c_ref,
                *, T, D, PAD, W, scale):
    qi = pl.program_id(1)
    i0 = qi * T

    @pl.when(qi == 0)
    def _():
        c_ref[...] = jnp.zeros_like(c_ref)

    qb = q_ref[0]
    kw = k_ref[0, pl.ds(i0 - LOCAL_HALF + PAD, W), :]
    vw = v_ref[0, pl.ds(i0 - LOCAL_HALF + PAD, W), :]
    s = jax.lax.dot_general(qb, kw, (((1,), (1,)), ((), ())),
                            preferred_element_type=jnp.float32)
    rt = jax.lax.broadcasted_iota(jnp.int32, (T, W), 0)
    cj = jax.lax.broadcasted_iota(jnp.int32, (T, W), 1)
    band = (cj >= rt) & (cj <= rt + LOCAL_HALF)
    w = jnp.where(band, jnp.exp(scale * s) - 1.0, 0.0)
    num = jax.lax.dot(w, vw, preferred_element_type=jnp.float32)
    den = jnp.sum(w, axis=1, keepdims=True)

    vb = v_ref[0, pl.ds(i0 + PAD, T), :]
    ri = jax.lax.broadcasted_iota(jnp.int32, (T, T), 0)
    ci = jax.lax.broadcasted_iota(jnp.int32, (T, T), 1)
    tril = (ri >= ci).astype(jnp.float32)
    p = jax.lax.dot(tril, vb, preferred_element_type=jnp.float32)
    carry = c_ref[...]
    c_ref[...] = carry + p[T - 1:T, :]

    rows = jax.lax.broadcasted_iota(jnp.int32, (T, 1), 0) + i0
    o_ref[0] = (carry + p + num + nb_ref[0]) / (
        (rows + 1).astype(jnp.float32) + den + db_ref[0])


def kernel(queries, keys, values, attn_mask):
    B, L, H, D = queries.shape
    del attn_mask
    scale = 1.0 / sqrt(D)
    BH = B * H

    qt = jnp.transpose(queries, (0, 2, 1, 3)).reshape(BH, L, D)
    kt = jnp.transpose(keys, (0, 2, 1, 3)).reshape(BH, L, D)
    vt = jnp.transpose(values, (0, 2, 1, 3)).reshape(BH, L, D)

    A = -(-L // SP1)
    Lp = SP1 * A
    G = 4
    R = -(-SP1 // G) * G
    def to_r(x):
        xp = jnp.pad(x, ((0, 0), (0, Lp - L), (0, 0)))
        xr = jnp.transpose(xp.reshape(BH, A, SP1, D), (0, 2, 1, 3))
        return jnp.pad(xr, ((0, 0), (0, R - SP1), (0, 0), (0, 0))
                       ).reshape(BH, R * A, D)
    qr, kr, vr = to_r(qt), to_r(kt), to_r(vt)
    n = G * A
    sb = functools.partial(_strided_body, G=G, A=A, scale=scale)
    num_r, den_r = pl.pallas_call(
        sb,
        grid=(BH, R // G),
        in_specs=[pl.BlockSpec((1, n, D), lambda bh, g: (bh, g, 0))] * 3,
        out_specs=[pl.BlockSpec((1, n, D), lambda bh, g: (bh, g, 0)),
                   pl.BlockSpec((1, n, 1), lambda bh, g: (bh, g, 0))],
        out_shape=[jax.ShapeDtypeStruct((BH, R * A, D), jnp.float32),
                   jax.ShapeDtypeStruct((BH, R * A, 1), jnp.float32)],
    )(qr, kr, vr)

    def from_r(x):
        d = x.shape[-1]
        xi = jnp.transpose(x.reshape(BH, R, A, d), (0, 2, 1, 3))[:, :, :SP1, :]
        return xi.reshape(BH, A * SP1, d)[:, :L, :]
    num_i, den_i = from_r(num_r), from_r(den_r)

    T = 128
    PAD = 128
    Wn = T + LOCAL_HALF
    zpad = jnp.zeros((BH, PAD, D), jnp.float32)
    kp = jnp.concatenate([zpad, kt], axis=1)
    vp = jnp.concatenate([zpad, vt], axis=1)
    lb = functools.partial(_local_body, T=T, D=D, PAD=PAD, W=Wn,
                           scale=scale)
    out = pl.pallas_call(
        lb,
        grid=(BH, L // T),
        in_specs=[
            pl.BlockSpec((1, T, D), lambda bh, qi: (bh, qi, 0)),
            pl.BlockSpec((1, PAD + L, D), lambda bh, qi: (bh, 0, 0)),
            pl.BlockSpec((1, PAD + L, D), lambda bh, qi: (bh, 0, 0)),
            pl.BlockSpec((1, T, D), lambda bh, qi: (bh, qi, 0)),
            pl.BlockSpec((1, T, 1), lambda bh, qi: (bh, qi, 0)),
        ],
        out_specs=pl.BlockSpec((1, T, D), lambda bh, qi: (bh, qi, 0)),
        out_shape=jax.ShapeDtypeStruct((BH, L, D), jnp.float32),
        scratch_shapes=[pltpu.VMEM((1, D), jnp.float32)],
    )(qt, kp, vp, num_i, den_i)
    return jnp.transpose(out.reshape(B, H, L, D), (0, 2, 1, 3))


# speedup vs baseline: 4.7552x; 4.7552x over previous
import functools
from math import sqrt

import jax
import jax.numpy as jnp
from jax.experimental import pallas as pl

LOCAL_HALF = 8
SP1 = 65


def _body(q_ref, k_ref, v_ref, o_ref, *, T, L, NH, D, scale):
    C = NH * D
    nblk = L // T
    f32 = jnp.float32

    ri = jax.lax.broadcasted_iota(jnp.int32, (T, T), 0)
    ci = jax.lax.broadcasted_iota(jnp.int32, (T, T), 1)
    tril = (ri >= ci).astype(f32)
    li = jax.lax.broadcasted_iota(jnp.int32, (C, NH), 0)
    hi = jax.lax.broadcasted_iota(jnp.int32, (C, NH), 1)
    sel = (li // D == hi).astype(f32)
    selt = jnp.transpose(sel)

    carry = jnp.zeros((1, C), f32)
    for blk in range(nblk):
        i0 = blk * T
        qb = q_ref[0, i0:i0 + T, :]
        vb = v_ref[0, i0:i0 + T, :]
        p = jax.lax.dot(tril, vb, preferred_element_type=f32) + carry
        carry = p[T - 1:T, :]

        diags = [d for d in range(LOCAL_HALF + 1)] + [
            SP1 * m for m in range(1, L // SP1 + 1)
            if SP1 * m <= i0 + T - 1]
        pieces = []
        vslices = []
        for d in diags:
            if d <= i0:
                ks = k_ref[0, i0 - d:i0 - d + T, :]
                vs = v_ref[0, i0 - d:i0 - d + T, :]
            else:
                sh = d - i0
                z = jnp.zeros((sh, C), f32)
                ks = jnp.concatenate([z, k_ref[0, 0:T - sh, :]], axis=0)
                vs = jnp.concatenate([z, v_ref[0, 0:T - sh, :]], axis=0)
            pieces.append(jax.lax.dot(qb * ks, sel,
                                      preferred_element_type=f32))
            vslices.append(vs)

        nd = len(diags)
        s_all = jnp.concatenate(pieces, axis=1)
        lane = jax.lax.broadcasted_iota(jnp.int32, (T, NH * nd), 1)
        didx = lane // NH
        dval = jnp.where(didx <= LOCAL_HALF, didx,
                         SP1 * (didx - LOCAL_HALF))
        rows = jax.lax.broadcasted_iota(jnp.int32, (T, NH * nd), 0) + i0
        w = jnp.where(rows >= dval,
                      jnp.exp(scale * s_all) - 1.0, 0.0)

        wl = jax.lax.broadcasted_iota(jnp.int32, (NH * nd, NH), 0)
        wh = jax.lax.broadcasted_iota(jnp.int32, (NH * nd, NH), 1)
        sumsel = (wl % NH == wh).astype(f32)
        den4 = jax.lax.dot(w, sumsel, preferred_element_type=f32)
        denb = jax.lax.dot(den4, selt, preferred_element_type=f32)

        num = jnp.zeros((T, C), f32)
        for j in range(nd):
            w4 = w[:, NH * j:NH * (j + 1)]
            wb = jax.lax.dot(w4, selt, preferred_element_type=f32)
            num = num + wb * vslices[j]

        cnt = (jax.lax.broadcasted_iota(jnp.int32, (T, 1), 0)
               + (i0 + 1)).astype(f32)
        o_ref[0, i0:i0 + T, :] = (p + num) / (cnt + denb)


def kernel(queries, keys, values, attn_mask):
    B, L, H, D = queries.shape
    del attn_mask
    scale = 1.0 / sqrt(D)
    NH = 4 if H % 4 == 0 else (2 if H % 2 == 0 else 1)
    C = NH * D
    HS = H // NH
    T = 256

    qv = queries.reshape(B, L, H * D)
    kv = keys.reshape(B, L, H * D)
    vv = values.reshape(B, L, H * D)

    body = functools.partial(_body, T=T, L=L, NH=NH, D=D, scale=scale)
    out = pl.pallas_call(
        body,
        grid=(B, HS),
        in_specs=[pl.BlockSpec((1, L, C), lambda b, hs: (b, 0, hs))] * 3,
        out_specs=pl.BlockSpec((1, L, C), lambda b, hs: (b, 0, hs)),
        out_shape=jax.ShapeDtypeStruct((B, L, H * D), jnp.float32),
    )(qv, kv, vv)
    return out.reshape(B, L, H, D)
